# baseline (speedup 1.0000x reference)
import functools

import jax
import jax.numpy as jnp
from jax import lax
from jax.experimental import pallas as pl
from jax.experimental.pallas import tpu as pltpu
from jax.experimental.pallas import tpu_sc as plsc

EMBED = 32
LANES = 16
CHUNK = 64
PROJ_BLK = 65536


def _proj_body(w_ref, tab_ref, out_ref):
    out_ref[...] = jnp.dot(
        w_ref[...], tab_ref[...], preferred_element_type=jnp.float32)[0]


def _tc_project(w_row, tab_t):
    n = tab_t.shape[1]
    grid = pl.cdiv(n, PROJ_BLK)
    return pl.pallas_call(
        _proj_body,
        grid=(grid,),
        in_specs=[
            pl.BlockSpec((1, EMBED), lambda i: (0, 0)),
            pl.BlockSpec((EMBED, PROJ_BLK), lambda i: (0, i)),
        ],
        out_specs=pl.BlockSpec((PROJ_BLK,), lambda i: (i,)),
        out_shape=jax.ShapeDtypeStruct((n,), jnp.float32),
    )(w_row, tab_t)


def _to_hbm(x):
    return pltpu.with_memory_space_constraint(x, pltpu.MemorySpace.HBM)


def _tc_project_from(w_row, tab_t, blk0):
    n = tab_t.shape[1]
    grid = pl.cdiv(n, PROJ_BLK) - blk0
    return pl.pallas_call(
        _proj_body,
        grid=(grid,),
        in_specs=[
            pl.BlockSpec((1, EMBED), lambda i: (0, 0)),
            pl.BlockSpec((EMBED, PROJ_BLK), lambda i: (0, i + blk0)),
        ],
        out_specs=pl.BlockSpec((PROJ_BLK,), lambda i: (i + blk0,)),
        out_shape=jax.ShapeDtypeStruct((n,), jnp.float32),
    )(w_row, tab_t)


SC_COLS = 196608


def _sc_proj_body(nc, cols_pw, w_h, tab_h, out_h, wv, slab, out_v, sem_w, sem):
    wid = lax.axis_index("s") * nc + lax.axis_index("c")
    base = wid * cols_pw
    csz = 512
    nch = cols_pw // csz

    pltpu.async_copy(w_h, wv, sem_w).wait()

    def chunk(c, carry):
        off = base + c * csz
        pltpu.async_copy(tab_h.at[:, pl.ds(off, csz)], slab, sem).wait()
        for m in range(csz // LANES):
            acc = slab[0, pl.ds(m * LANES, LANES)] * wv[0, pl.ds(0, LANES)]
            for d in range(1, EMBED):
                acc = acc + (slab[d, pl.ds(m * LANES, LANES)]
                             * wv[d, pl.ds(0, LANES)])
            out_v[pl.ds(c * csz + m * LANES, LANES)] = acc
        return carry

    lax.fori_loop(0, nch, chunk, 0)
    pltpu.sync_copy(out_v, out_h.at[pl.ds(base, cols_pw)])


def _sc_project_prefix(w_splat, tab_t):
    info = plsc.get_sparse_core_info()
    nc, ns = info.num_cores, info.num_subcores
    cols_pw = SC_COLS // (nc * ns)

    mesh = plsc.VectorSubcoreMesh(core_axis_name="c", subcore_axis_name="s")
    fn = pl.kernel(
        functools.partial(_sc_proj_body, nc, cols_pw),
        out_type=jax.ShapeDtypeStruct((SC_COLS,), jnp.float32),
        mesh=mesh,
        compiler_params=pltpu.CompilerParams(
            needs_layout_passes=False, use_tc_tiling_on_sc=True),
        scratch_types=[
            pltpu.VMEM((EMBED, 128), jnp.float32),
            pltpu.VMEM((EMBED, 512), jnp.float32),
            pltpu.VMEM((cols_pw,), jnp.float32),
            pltpu.SemaphoreType.DMA,
            pltpu.SemaphoreType.DMA,
        ],
    )
    return fn(w_splat, tab_t)


def _sc_body(nc, bpw, idx_h, proj_h, bv_h, out_h,
             idx_v, g_v, bv_v, out_v, sem_i, sem_b, sem):
    wid = lax.axis_index("s") * nc + lax.axis_index("c")
    base = wid * bpw
    nch = bpw // CHUNK
    elementwise = bv_h.shape[0] != LANES

    idx_cp = pltpu.async_copy(idx_h.at[pl.ds(base, bpw)], idx_v, sem_i)
    if elementwise:
        bv_cp = pltpu.async_copy(bv_h.at[pl.ds(base, bpw)], bv_v, sem_b)
    else:
        bv_cp = pltpu.async_copy(bv_h, bv_v, sem_b)

    idx_cp.wait()
    copies = []
    for j in range(nch):
        copies.append(pltpu.async_copy(
            proj_h.at[idx_v.at[pl.ds(j * CHUNK, CHUNK)]],
            g_v.at[pl.ds(j * CHUNK, CHUNK)], sem))
    bv_cp.wait()
    for c in copies:
        c.wait()

    for i in range(0, bpw, LANES):
        bval = bv_v[pl.ds(i, LANES)] if elementwise else bv_v[...]
        out_v[pl.ds(i, LANES)] = g_v[pl.ds(i, LANES)] + bval

    pltpu.sync_copy(out_v, out_h.at[pl.ds(base, bpw)])


def _sc_gather_add(idx, proj, base_vals):
    batch = idx.shape[0]
    info = plsc.get_sparse_core_info()
    nc, ns = info.num_cores, info.num_subcores
    bpw = batch // (nc * ns)

    bv_shape = (LANES,) if base_vals.shape[0] == LANES else (bpw,)
    mesh = plsc.VectorSubcoreMesh(core_axis_name="c", subcore_axis_name="s")
    fn = pl.kernel(
        functools.partial(_sc_body, nc, bpw),
        out_type=jax.ShapeDtypeStruct((batch,), jnp.float32),
        mesh=mesh,
        compiler_params=pltpu.CompilerParams(
            needs_layout_passes=False, use_tc_tiling_on_sc=False),
        scratch_types=[
            pltpu.VMEM((bpw,), jnp.int32),
            pltpu.VMEM((bpw,), jnp.float32),
            pltpu.VMEM(bv_shape, jnp.float32),
            pltpu.VMEM((bpw,), jnp.float32),
            pltpu.SemaphoreType.DMA,
            pltpu.SemaphoreType.DMA,
            pltpu.SemaphoreType.DMA,
        ],
    )
    return fn(idx, proj, base_vals)


def _sc_body2(nc, bpw, thresh, idx_h, plo_h, phi_h, bv_h, out_h,
              idx_v, ilo_v, g_lo, g_hi, bv_v, out_v, sem_i, sem_b, sem):
    wid = lax.axis_index("s") * nc + lax.axis_index("c")
    base = wid * bpw
    nch = bpw // CHUNK

    idx_cp = pltpu.async_copy(idx_h.at[pl.ds(base, bpw)], idx_v, sem_i)
    bv_cp = pltpu.async_copy(bv_h.at[pl.ds(base, bpw)], bv_v, sem_b)

    idx_cp.wait()
    for i in range(0, bpw, LANES):
        ilo_v[pl.ds(i, LANES)] = jnp.minimum(idx_v[pl.ds(i, LANES)],
                                             thresh - 1)
    copies = []
    for j in range(nch):
        copies.append(pltpu.async_copy(
            plo_h.at[ilo_v.at[pl.ds(j * CHUNK, CHUNK)]],
            g_lo.at[pl.ds(j * CHUNK, CHUNK)], sem))
        copies.append(pltpu.async_copy(
            phi_h.at[idx_v.at[pl.ds(j * CHUNK, CHUNK)]],
            g_hi.at[pl.ds(j * CHUNK, CHUNK)], sem))
    bv_cp.wait()
    for c in copies:
        c.wait()

    for i in range(0, bpw, LANES):
        sel = idx_v[pl.ds(i, LANES)] < thresh
        val = jnp.where(sel, g_lo[pl.ds(i, LANES)], g_hi[pl.ds(i, LANES)])
        out_v[pl.ds(i, LANES)] = val + bv_v[pl.ds(i, LANES)]

    pltpu.sync_copy(out_v, out_h.at[pl.ds(base, bpw)])


def _sc_gather_add2(idx, proj_lo, proj_hi, base_vals, thresh):
    batch = idx.shape[0]
    info = plsc.get_sparse_core_info()
    nc, ns = info.num_cores, info.num_subcores
    bpw = batch // (nc * ns)

    mesh = plsc.VectorSubcoreMesh(core_axis_name="c", subcore_axis_name="s")
    fn = pl.kernel(
        functools.partial(_sc_body2, nc, bpw, thresh),
        out_type=jax.ShapeDtypeStruct((batch,), jnp.float32),
        mesh=mesh,
        compiler_params=pltpu.CompilerParams(
            needs_layout_passes=False, use_tc_tiling_on_sc=False),
        scratch_types=[
            pltpu.VMEM((bpw,), jnp.int32),
            pltpu.VMEM((bpw,), jnp.int32),
            pltpu.VMEM((bpw,), jnp.float32),
            pltpu.VMEM((bpw,), jnp.float32),
            pltpu.VMEM((bpw,), jnp.float32),
            pltpu.VMEM((bpw,), jnp.float32),
            pltpu.SemaphoreType.DMA,
            pltpu.SemaphoreType.DMA,
            pltpu.SemaphoreType.DMA,
        ],
    )
    return fn(idx, proj_lo, proj_hi, base_vals)


@jax.jit
def _run(user, course, user_table, course_table, W, b):
    w_u = W[:, :EMBED]
    w_c = W[:, EMBED:]
    b_vec = jnp.broadcast_to(b, (LANES,)).astype(jnp.float32)
    w_u_splat = jnp.broadcast_to(
        W[0, :EMBED].reshape(EMBED, 1), (EMBED, 128)).astype(jnp.float32)
    proj_c = _to_hbm(_tc_project(w_c, course_table.T))
    partial = _sc_gather_add(course, proj_c, b_vec)
    sc_lo = _to_hbm(_sc_project_prefix(w_u_splat, user_table.T))
    proj_u_hi = _to_hbm(_tc_project_from(w_u, user_table.T, SC_COLS // PROJ_BLK))
    return _sc_gather_add2(user, sc_lo, proj_u_hi, partial, SC_COLS)


def kernel(user, course, user_table, course_table, W, b):
    out = _run(user, course, user_table, course_table, W, b)
    return out.reshape(-1, 1)


# speedup vs baseline: 2.4972x; 2.4972x over previous
import functools

import jax
import jax.numpy as jnp
from jax import lax
from jax.experimental import pallas as pl
from jax.experimental.pallas import tpu as pltpu
from jax.experimental.pallas import tpu_sc as plsc

EMBED = 32
LANES = 16
CHUNK = 64
PROJ_BLK = 65536


def _proj_body(w_ref, tab_ref, out_ref):
    out_ref[...] = jnp.dot(
        w_ref[...], tab_ref[...], preferred_element_type=jnp.float32)[0]


def _tc_project(w_row, tab_t):
    n = tab_t.shape[1]
    grid = pl.cdiv(n, PROJ_BLK)
    return pl.pallas_call(
        _proj_body,
        grid=(grid,),
        in_specs=[
            pl.BlockSpec((1, EMBED), lambda i: (0, 0)),
            pl.BlockSpec((EMBED, PROJ_BLK), lambda i: (0, i)),
        ],
        out_specs=pl.BlockSpec((PROJ_BLK,), lambda i: (i,)),
        out_shape=jax.ShapeDtypeStruct((n,), jnp.float32),
    )(w_row, tab_t)


def _to_hbm(x):
    return pltpu.with_memory_space_constraint(x, pltpu.MemorySpace.HBM)


def _sc_body(nc, bpw, idx_h, proj_h, bv_h, out_h,
             idx_v, g_v, bv_v, out_v, sem_i, sem_b, sem):
    wid = lax.axis_index("s") * nc + lax.axis_index("c")
    base = wid * bpw
    nch = bpw // CHUNK
    elementwise = bv_h.shape[0] != LANES

    idx_cp = pltpu.async_copy(idx_h.at[pl.ds(base, bpw)], idx_v, sem_i)
    if elementwise:
        bv_cp = pltpu.async_copy(bv_h.at[pl.ds(base, bpw)], bv_v, sem_b)
    else:
        bv_cp = pltpu.async_copy(bv_h, bv_v, sem_b)

    idx_cp.wait()
    copies = []
    for j in range(nch):
        copies.append(pltpu.async_copy(
            proj_h.at[idx_v.at[pl.ds(j * CHUNK, CHUNK)]],
            g_v.at[pl.ds(j * CHUNK, CHUNK)], sem))
    bv_cp.wait()
    for c in copies:
        c.wait()

    for i in range(0, bpw, LANES):
        bval = bv_v[pl.ds(i, LANES)] if elementwise else bv_v[...]
        out_v[pl.ds(i, LANES)] = g_v[pl.ds(i, LANES)] + bval

    pltpu.sync_copy(out_v, out_h.at[pl.ds(base, bpw)])


def _sc_gather_add(idx, proj, base_vals):
    batch = idx.shape[0]
    info = plsc.get_sparse_core_info()
    nc, ns = info.num_cores, info.num_subcores
    bpw = batch // (nc * ns)

    bv_shape = (LANES,) if base_vals.shape[0] == LANES else (bpw,)
    mesh = plsc.VectorSubcoreMesh(core_axis_name="c", subcore_axis_name="s")
    fn = pl.kernel(
        functools.partial(_sc_body, nc, bpw),
        out_type=jax.ShapeDtypeStruct((batch,), jnp.float32),
        mesh=mesh,
        compiler_params=pltpu.CompilerParams(
            needs_layout_passes=False, use_tc_tiling_on_sc=False),
        scratch_types=[
            pltpu.VMEM((bpw,), jnp.int32),
            pltpu.VMEM((bpw,), jnp.float32),
            pltpu.VMEM(bv_shape, jnp.float32),
            pltpu.VMEM((bpw,), jnp.float32),
            pltpu.SemaphoreType.DMA,
            pltpu.SemaphoreType.DMA,
            pltpu.SemaphoreType.DMA,
        ],
    )
    return fn(idx, proj, base_vals)


@jax.jit
def _run(user, course, user_table, course_table, W, b):
    w_u = W[:, :EMBED]
    w_c = W[:, EMBED:]
    b_vec = jnp.broadcast_to(b, (LANES,)).astype(jnp.float32)
    proj_c = _to_hbm(_tc_project(w_c, course_table.T))
    partial = _sc_gather_add(course, proj_c, b_vec)
    proj_u = _to_hbm(_tc_project(w_u, user_table.T))
    return _sc_gather_add(user, proj_u, partial)


def kernel(user, course, user_table, course_table, W, b):
    out = _run(user, course, user_table, course_table, W, b)
    return out.reshape(-1, 1)
